# baseline (speedup 1.0000x reference)
import jax
import jax.numpy as jnp
from jax import lax
from jax.experimental import pallas as pl
from jax.experimental.pallas import tpu as pltpu
from jax.experimental.pallas import tpu_sc as plsc

EPS = 1e-07
NCE_T = 0.07
N_DATA = 50000
P_POS = 4
BSZ = 512
K_TOT = 1028
FEAT = 128
NUM_CLS = 100
PER_CLS = 500

NC, NS, L = 2, 16, 16
NW = NC * NS
A_T = BSZ // NW
E_T = A_T * K_TOT
N_CHUNK_FULL = E_T // 128
TAIL = E_T - N_CHUNK_FULL * 128
R_BLK = 1000
N_RSTEP = N_DATA // R_BLK
S_ROWS = BSZ * N_DATA // 128

_HI = jax.lax.Precision.HIGHEST


def _embed_body(fs_ref, ws_ref, bs_ref, ft_ref, wt_ref, bt_ref, es_ref, et_ref):
    def emb(f, w, b):
        dims = (((1,), (1,)), ((), ()))
        f16 = f.astype(jnp.bfloat16)
        w16 = w.astype(jnp.bfloat16)
        flo = (f - f16.astype(jnp.float32)).astype(jnp.bfloat16)
        wlo = (w - w16.astype(jnp.float32)).astype(jnp.bfloat16)
        x = (lax.dot_general(f16, w16, dims, preferred_element_type=jnp.float32)
             + lax.dot_general(f16, wlo, dims, preferred_element_type=jnp.float32)
             + lax.dot_general(flo, w16, dims, preferred_element_type=jnp.float32))
        x = x + b
        inv = lax.rsqrt(jnp.sum(x * x, axis=1, keepdims=True))
        return x * (inv * (1.0 / NCE_T))

    es_ref[...] = emb(fs_ref[...], ws_ref[...], bs_ref[...])
    et_ref[...] = emb(ft_ref[...], wt_ref[...], bt_ref[...])


def _embed(f_s, W_s, b_s, f_t, W_t, b_t):
    return pl.pallas_call(
        _embed_body,
        out_shape=(jax.ShapeDtypeStruct((BSZ, FEAT), jnp.float32),
                   jax.ShapeDtypeStruct((BSZ, FEAT), jnp.float32)),
    )(f_s, W_s, b_s.reshape(1, FEAT), f_t, W_t, b_t.reshape(1, FEAT))


def _scores_body(ms_ref, mt_ref, es_ref, et_ref, spk_ref):
    cs = lax.dot_general(mt_ref[...], es_ref[...], (((1,), (1,)), ((), ())),
                         preferred_element_type=jnp.float32)
    ct = lax.dot_general(ms_ref[...], et_ref[...], (((1,), (1,)), ((), ())),
                         preferred_element_type=jnp.float32)
    us = lax.bitcast_convert_type(cs, jnp.uint32)
    ut = lax.bitcast_convert_type(ct, jnp.uint32)
    us = us + jnp.uint32(0x8000)
    ut = ut + jnp.uint32(0x8000)
    packed = lax.bitcast_convert_type(
        lax.shift_right_logical(us, jnp.uint32(16))
        | (ut & jnp.uint32(0xFFFF0000)), jnp.float32)
    for g in range(4):
        spk_ref[pl.ds(g * R_BLK, R_BLK), :] = packed[:, g * 128:(g + 1) * 128]


def _scores(memory_s, memory_t, emb_s, emb_t):
    blk = pl.BlockSpec((R_BLK, FEAT), lambda i: (i, 0))
    full = pl.BlockSpec((BSZ, FEAT), lambda i: (0, 0))
    out_blk = pl.BlockSpec((4 * R_BLK, 128), lambda i: (i, 0))
    return pl.pallas_call(
        _scores_body,
        grid=(N_RSTEP,),
        in_specs=[blk, blk, full, full],
        out_specs=out_blk,
        out_shape=jax.ShapeDtypeStruct((S_ROWS, 128), jnp.float32),
    )(memory_s, memory_t, emb_s, emb_t)


def _sc_scores_body(spk_hbm, fidx_hbm, sums_hbm, pos_hbm,
                    idx_v, val_v, out16_v, pos_v, pos2_v,
                    gsem, gsem2, gsem3, gsem4):
    wid = lax.axis_index("c") * NS + lax.axis_index("s")
    pltpu.sync_copy(fidx_hbm.at[wid], idx_v)
    NBP = 6
    qsems = (gsem, gsem2, gsem3, gsem4)

    def fire(sem, c):
        pltpu.async_copy(spk_hbm.at[idx_v.at[c]],
                         val_v.at[pl.ds(c * 128, 128)], sem)

    def wait(sem, c):
        pltpu.make_async_copy(spk_hbm.at[idx_v.at[c]],
                              val_v.at[pl.ds(c * 128, 128)], sem).wait()

    for cp in range(NBP):
        for q in range(4):
            fire(qsems[q], 4 * cp + q)

    def body(i, _):
        for q in range(4):
            wait(qsems[q], 4 * i + q)

        @pl.when(i < N_CHUNK_FULL // 4 - NBP)
        def _():
            for q in range(4):
                fire(qsems[q], 4 * (i + NBP) + q)
        return 0

    lax.fori_loop(0, N_CHUNK_FULL // 4, body, 0)
    pltpu.async_copy(spk_hbm.at[idx_v.at[N_CHUNK_FULL, pl.ds(0, TAIL)]],
                     val_v.at[pl.ds(N_CHUNK_FULL * 128, TAIL)], gsem)
    pltpu.make_async_copy(
        spk_hbm.at[idx_v.at[N_CHUNK_FULL, pl.ds(0, TAIL)]],
        val_v.at[pl.ds(N_CHUNK_FULL * 128, TAIL)], gsem).wait()

    def unpack(v):
        u = lax.bitcast_convert_type(v, jnp.int32)
        lo = lax.bitcast_convert_type(u << 16, jnp.float32)
        hi = lax.bitcast_convert_type(u & jnp.int32(-65536), jnp.float32)
        return lo, hi

    def group(base):
        return unpack(val_v[pl.ds(base, 16)])

    def body2(i, carry):
        acc_s, acc_t = carry
        base = i * 128
        for gg in range(8):
            lo, hi = group(base + gg * 16)
            acc_s = acc_s + jnp.exp(lo)
            acc_t = acc_t + jnp.exp(hi)
        return (acc_s, acc_t)

    z16 = jnp.zeros((16,), jnp.float32)
    acc_s, acc_t = lax.fori_loop(0, N_CHUNK_FULL, body2, (z16, z16))
    base = N_CHUNK_FULL * 128
    for gg in range(TAIL // 16):
        lo, hi = group(base + gg * 16)
        acc_s = acc_s + jnp.exp(lo)
        acc_t = acc_t + jnp.exp(hi)

    out16_v[...] = acc_s
    pltpu.sync_copy(out16_v, sums_hbm.at[0, wid])
    out16_v[...] = acc_t
    pltpu.sync_copy(out16_v, sums_hbm.at[1, wid])

    for a in range(A_T):
        lo, hi = group(a * K_TOT)
        pos_v[pl.ds(a * 16, 16)] = jnp.exp(lo)
        pos2_v[pl.ds(a * 16, 16)] = jnp.exp(hi)
    pltpu.sync_copy(pos_v, pos_hbm.at[0, wid])
    pltpu.sync_copy(pos2_v, pos_hbm.at[1, wid])


def _sc_scores(spk_flat, fidx):
    mesh = plsc.VectorSubcoreMesh(core_axis_name="c", subcore_axis_name="s")
    kfn = pl.kernel(
        _sc_scores_body,
        out_type=(jax.ShapeDtypeStruct((2, NW, 16), jnp.float32),
                  jax.ShapeDtypeStruct((2, NW, A_T * 16), jnp.float32)),
        mesh=mesh,
        scratch_types=[
            pltpu.VMEM((N_CHUNK_FULL + 1, 128), jnp.int32),
            pltpu.VMEM(((N_CHUNK_FULL + 1) * 128,), jnp.float32),
            pltpu.VMEM((16,), jnp.float32),
            pltpu.VMEM((A_T * 16,), jnp.float32),
            pltpu.VMEM((A_T * 16,), jnp.float32),
            pltpu.SemaphoreType.DMA,
            pltpu.SemaphoreType.DMA,
            pltpu.SemaphoreType.DMA,
            pltpu.SemaphoreType.DMA,
        ],
    )
    return kfn(spk_flat, fidx)


def _sc_anchors_body(cidx_hbm, ms_hbm, mt_hbm, anch_hbm,
                     cls_v, rows_v, anch_v, csem):
    wid = lax.axis_index("c") * NS + lax.axis_index("s")

    for rep in range(4):
        @pl.when(wid + rep * NW < NUM_CLS)
        def _():
            pltpu.sync_copy(cidx_hbm.at[wid + rep * NW], cls_v.at[rep])

    def fire(m_hbm, rep, j):
        pltpu.async_copy(m_hbm.at[cls_v.at[rep, j]],
                         rows_v.at[pl.ds(j * 128, 128)], csem)

    def wait_chunk(m_hbm, rep, j):
        pltpu.make_async_copy(m_hbm.at[cls_v.at[rep, j]],
                              rows_v.at[pl.ds(j * 128, 128)], csem).wait()

    for bank, m_hbm in ((0, ms_hbm), (1, mt_hbm)):
        for j in range(4):
            fire(m_hbm, 0, j)

        def rep_body(rep, _):
            cls = wid + rep * NW
            cls_ok = cls < NUM_CLS
            rep1 = jnp.minimum(rep + 1, 3)
            nxt_ok = jnp.logical_and(rep + 1 < 4,
                                     wid + (rep + 1) * NW < NUM_CLS)
            carry = tuple(jnp.zeros((16,), jnp.float32) for _ in range(8))
            for j in range(4):
                @pl.when(cls_ok)
                def _():
                    wait_chunk(m_hbm, rep, j)

                def row_body(i, c):
                    c = list(c)
                    for u in range(8):
                        for gg in range(8):
                            c[gg] = c[gg] + jnp.maximum(
                                rows_v[j * 128 + i * 8 + u,
                                       pl.ds(gg * 16, 16)], 0.0)
                    return tuple(c)

                carry = lax.fori_loop(0, 16, row_body, carry)

                if j == 3:
                    carry = tuple(
                        carry[gg] - 12.0 * jnp.maximum(
                            rows_v[3 * 128 + 115, pl.ds(gg * 16, 16)], 0.0)
                        for gg in range(8))

                @pl.when(nxt_ok)
                def _():
                    fire(m_hbm, rep1, j)

            @pl.when(cls_ok)
            def _():
                for gg in range(8):
                    anch_v[pl.ds(gg * 16, 16)] = carry[gg]
                pltpu.sync_copy(anch_v, anch_hbm.at[bank, cls])
            return 0

        lax.fori_loop(0, 4, rep_body, 0)


def _sc_anchors(cidx, memory_s, memory_t):
    mesh = plsc.VectorSubcoreMesh(core_axis_name="c", subcore_axis_name="s")
    kfn = pl.kernel(
        _sc_anchors_body,
        out_type=jax.ShapeDtypeStruct((2, NUM_CLS, FEAT), jnp.float32),
        mesh=mesh,
        scratch_types=[
            pltpu.VMEM((4, 4, 128), jnp.int32),
            pltpu.VMEM((512, FEAT), jnp.float32),
            pltpu.VMEM((FEAT,), jnp.float32),
            pltpu.SemaphoreType.DMA,
        ],
    )
    return kfn(cidx, memory_s, memory_t)


def _finalize_body(sums_ref, pos_ref, anch_ref, es_ref, et_ref, ccd_ref, rel_ref):
    n_neg_c = (K_TOT - P_POS) * (1.0 / N_DATA) + EPS

    pmask = (lax.broadcasted_iota(jnp.int32, (NW, A_T * 16), 1) % 16) < P_POS

    def closs(bank):
        z = jnp.sum(sums_ref[bank]) * (float(N_DATA) / (BSZ * K_TOT))
        pn = pos_ref[bank] / z
        terms = jnp.log(pn / (pn + n_neg_c))
        return -jnp.sum(jnp.where(pmask, terms, 0.0)) / BSZ

    ccd_ref[...] = jnp.reshape(closs(0) + closs(1), (1, 1))

    def relation(emb, bank):
        a = anch_ref[bank] * (1.0 / PER_CLS)
        a = a * lax.rsqrt(jnp.sum(a * a, axis=1, keepdims=True))
        return lax.dot_general(emb, a, (((1,), (1,)), ((), ())),
                               preferred_element_type=jnp.float32,
                               precision=_HI)

    s_rel = relation(es_ref[...], 0)
    t_rel = relation(et_ref[...], 1)

    def logsoftmax(x):
        m = jnp.max(x, axis=1, keepdims=True)
        s = x - m
        return s - jnp.log(jnp.sum(jnp.exp(s), axis=1, keepdims=True))

    log_p_s = logsoftmax(s_rel)
    log_p_t = logsoftmax(t_rel)
    p_t = jnp.exp(log_p_t)
    rel_ref[...] = jnp.reshape(jnp.sum(p_t * (log_p_t - log_p_s)) * (1.0 / BSZ),
                               (1, 1))


def _finalize(sums, pos, anch, emb_s, emb_t):
    return pl.pallas_call(
        _finalize_body,
        out_shape=(jax.ShapeDtypeStruct((1, 1), jnp.float32),
                   jax.ShapeDtypeStruct((1, 1), jnp.float32)),
    )(sums, pos, anch, emb_s, emb_t)


def kernel(f_s, f_t, batch_label, class_index, num_pos, contrast_idx,
           W_s, b_s, W_t, b_t, memory_s, memory_t):
    ci32 = class_index.astype(jnp.int32)
    cidx = jnp.concatenate(
        [ci32, jnp.broadcast_to(ci32[:, -1:], (NUM_CLS, 12))], axis=1)
    cidx = cidx.reshape(NUM_CLS, 4, 128)
    anch = _sc_anchors(cidx, memory_s, memory_t)

    emb_s, emb_t = _embed(f_s, W_s, b_s, f_t, W_t, b_t)
    spk = _scores(memory_s, memory_t, emb_s, emb_t)

    r = contrast_idx.astype(jnp.int32)
    b = jnp.arange(BSZ, dtype=jnp.int32)[:, None]
    flat = ((r // R_BLK) * (R_BLK * BSZ) + (b // 128) * (R_BLK * 128)
            + (r % R_BLK) * 128 + (b % 128))
    flat = flat.reshape(NW, E_T)
    flat = jnp.pad(flat, ((0, 0), (0, 128 - TAIL))).reshape(NW, N_CHUNK_FULL + 1, 128)

    sums, pos = _sc_scores(
        spk.reshape(S_ROWS * 128), flat)

    ccd, rel = _finalize(sums, pos, anch, emb_s, emb_t)
    return (ccd[0, 0], rel[0, 0])


# speedup vs baseline: 1.1016x; 1.1016x over previous
import jax
import jax.numpy as jnp
from jax import lax
from jax.experimental import pallas as pl
from jax.experimental.pallas import tpu as pltpu
from jax.experimental.pallas import tpu_sc as plsc

EPS = 1e-07
NCE_T = 0.07
N_DATA = 50000
P_POS = 4
BSZ = 512
K_TOT = 1028
FEAT = 128
NUM_CLS = 100
PER_CLS = 500

NC, NS, L = 2, 16, 16
NW = NC * NS
A_T = BSZ // NW
E_T = A_T * K_TOT
N_CHUNK_FULL = E_T // 128
TAIL = E_T - N_CHUNK_FULL * 128
R_BLK = 2000
N_RSTEP = N_DATA // R_BLK
S_ROWS = BSZ * N_DATA // 128

_HI = jax.lax.Precision.HIGHEST


def _embed_body(fs_ref, ws_ref, bs_ref, ft_ref, wt_ref, bt_ref, es_ref, et_ref):
    def emb(f, w, b):
        dims = (((1,), (1,)), ((), ()))
        f16 = f.astype(jnp.bfloat16)
        w16 = w.astype(jnp.bfloat16)
        flo = (f - f16.astype(jnp.float32)).astype(jnp.bfloat16)
        wlo = (w - w16.astype(jnp.float32)).astype(jnp.bfloat16)
        x = (lax.dot_general(f16, w16, dims, preferred_element_type=jnp.float32)
             + lax.dot_general(f16, wlo, dims, preferred_element_type=jnp.float32)
             + lax.dot_general(flo, w16, dims, preferred_element_type=jnp.float32))
        x = x + b
        inv = lax.rsqrt(jnp.sum(x * x, axis=1, keepdims=True))
        return x * (inv * (1.0 / NCE_T))

    es_ref[...] = emb(fs_ref[...], ws_ref[...], bs_ref[...])
    et_ref[...] = emb(ft_ref[...], wt_ref[...], bt_ref[...])


def _embed(f_s, W_s, b_s, f_t, W_t, b_t):
    return pl.pallas_call(
        _embed_body,
        out_shape=(jax.ShapeDtypeStruct((BSZ, FEAT), jnp.float32),
                   jax.ShapeDtypeStruct((BSZ, FEAT), jnp.float32)),
    )(f_s, W_s, b_s.reshape(1, FEAT), f_t, W_t, b_t.reshape(1, FEAT))


def _scores_body(ms_ref, mt_ref, es_ref, et_ref, spk_ref):
    cs = lax.dot_general(mt_ref[...], es_ref[...], (((1,), (1,)), ((), ())),
                         preferred_element_type=jnp.float32)
    ct = lax.dot_general(ms_ref[...], et_ref[...], (((1,), (1,)), ((), ())),
                         preferred_element_type=jnp.float32)
    us = lax.bitcast_convert_type(cs, jnp.uint32)
    ut = lax.bitcast_convert_type(ct, jnp.uint32)
    us = us + jnp.uint32(0x8000)
    ut = ut + jnp.uint32(0x8000)
    packed = lax.bitcast_convert_type(
        lax.shift_right_logical(us, jnp.uint32(16))
        | (ut & jnp.uint32(0xFFFF0000)), jnp.float32)
    for g in range(4):
        spk_ref[pl.ds(g * R_BLK, R_BLK), :] = packed[:, g * 128:(g + 1) * 128]


def _scores(memory_s, memory_t, emb_s, emb_t):
    blk = pl.BlockSpec((R_BLK, FEAT), lambda i: (i, 0))
    full = pl.BlockSpec((BSZ, FEAT), lambda i: (0, 0))
    out_blk = pl.BlockSpec((4 * R_BLK, 128), lambda i: (i, 0))
    return pl.pallas_call(
        _scores_body,
        grid=(N_RSTEP,),
        in_specs=[blk, blk, full, full],
        out_specs=out_blk,
        out_shape=jax.ShapeDtypeStruct((S_ROWS, 128), jnp.float32),
    )(memory_s, memory_t, emb_s, emb_t)


def _sc_scores_body(spk_hbm, fidx_hbm, sums_hbm, pos_hbm,
                    idx_v, val_v, out16_v, pos_v, pos2_v,
                    gsem, gsem2, gsem3, gsem4):
    wid = lax.axis_index("c") * NS + lax.axis_index("s")
    pltpu.sync_copy(fidx_hbm.at[wid], idx_v)
    NBP = 6
    qsems = (gsem, gsem2, gsem3, gsem4)

    def fire(sem, c):
        pltpu.async_copy(spk_hbm.at[idx_v.at[c]],
                         val_v.at[pl.ds(c * 128, 128)], sem)

    def wait(sem, c):
        pltpu.make_async_copy(spk_hbm.at[idx_v.at[c]],
                              val_v.at[pl.ds(c * 128, 128)], sem).wait()

    for cp in range(NBP):
        for q in range(4):
            fire(qsems[q], 4 * cp + q)

    def body(i, _):
        for q in range(4):
            wait(qsems[q], 4 * i + q)

        @pl.when(i < N_CHUNK_FULL // 4 - NBP)
        def _():
            for q in range(4):
                fire(qsems[q], 4 * (i + NBP) + q)
        return 0

    lax.fori_loop(0, N_CHUNK_FULL // 4, body, 0)
    pltpu.async_copy(spk_hbm.at[idx_v.at[N_CHUNK_FULL, pl.ds(0, TAIL)]],
                     val_v.at[pl.ds(N_CHUNK_FULL * 128, TAIL)], gsem)
    pltpu.make_async_copy(
        spk_hbm.at[idx_v.at[N_CHUNK_FULL, pl.ds(0, TAIL)]],
        val_v.at[pl.ds(N_CHUNK_FULL * 128, TAIL)], gsem).wait()

    def unpack(v):
        u = lax.bitcast_convert_type(v, jnp.int32)
        lo = lax.bitcast_convert_type(u << 16, jnp.float32)
        hi = lax.bitcast_convert_type(u & jnp.int32(-65536), jnp.float32)
        return lo, hi

    def group(base):
        return unpack(val_v[pl.ds(base, 16)])

    def body2(i, carry):
        acc_s, acc_t = carry
        base = i * 128
        for gg in range(8):
            lo, hi = group(base + gg * 16)
            acc_s = acc_s + jnp.exp(lo)
            acc_t = acc_t + jnp.exp(hi)
        return (acc_s, acc_t)

    z16 = jnp.zeros((16,), jnp.float32)
    acc_s, acc_t = lax.fori_loop(0, N_CHUNK_FULL, body2, (z16, z16))
    base = N_CHUNK_FULL * 128
    for gg in range(TAIL // 16):
        lo, hi = group(base + gg * 16)
        acc_s = acc_s + jnp.exp(lo)
        acc_t = acc_t + jnp.exp(hi)

    out16_v[...] = acc_s
    pltpu.sync_copy(out16_v, sums_hbm.at[0, wid])
    out16_v[...] = acc_t
    pltpu.sync_copy(out16_v, sums_hbm.at[1, wid])

    for a in range(A_T):
        lo, hi = group(a * K_TOT)
        pos_v[pl.ds(a * 16, 16)] = jnp.exp(lo)
        pos2_v[pl.ds(a * 16, 16)] = jnp.exp(hi)
    pltpu.sync_copy(pos_v, pos_hbm.at[0, wid])
    pltpu.sync_copy(pos2_v, pos_hbm.at[1, wid])


def _sc_scores(spk_flat, fidx):
    mesh = plsc.VectorSubcoreMesh(core_axis_name="c", subcore_axis_name="s")
    kfn = pl.kernel(
        _sc_scores_body,
        out_type=(jax.ShapeDtypeStruct((2, NW, 16), jnp.float32),
                  jax.ShapeDtypeStruct((2, NW, A_T * 16), jnp.float32)),
        mesh=mesh,
        scratch_types=[
            pltpu.VMEM((N_CHUNK_FULL + 1, 128), jnp.int32),
            pltpu.VMEM(((N_CHUNK_FULL + 1) * 128,), jnp.float32),
            pltpu.VMEM((16,), jnp.float32),
            pltpu.VMEM((A_T * 16,), jnp.float32),
            pltpu.VMEM((A_T * 16,), jnp.float32),
            pltpu.SemaphoreType.DMA,
            pltpu.SemaphoreType.DMA,
            pltpu.SemaphoreType.DMA,
            pltpu.SemaphoreType.DMA,
        ],
    )
    return kfn(spk_flat, fidx)


def _sc_anchors_body(cidx_hbm, ms_hbm, mt_hbm, anch_hbm,
                     cls_v, rows_v, anch_v, csem):
    wid = lax.axis_index("c") * NS + lax.axis_index("s")

    for rep in range(4):
        @pl.when(wid + rep * NW < NUM_CLS)
        def _():
            pltpu.sync_copy(cidx_hbm.at[wid + rep * NW], cls_v.at[rep])

    def fire(m_hbm, rep, j):
        pltpu.async_copy(m_hbm.at[cls_v.at[rep, j]],
                         rows_v.at[pl.ds(j * 128, 128)], csem)

    def wait_chunk(m_hbm, rep, j):
        pltpu.make_async_copy(m_hbm.at[cls_v.at[rep, j]],
                              rows_v.at[pl.ds(j * 128, 128)], csem).wait()

    for bank, m_hbm in ((0, ms_hbm), (1, mt_hbm)):
        for j in range(4):
            fire(m_hbm, 0, j)

        def rep_body(rep, _):
            cls = wid + rep * NW
            cls_ok = cls < NUM_CLS
            rep1 = jnp.minimum(rep + 1, 3)
            nxt_ok = jnp.logical_and(rep + 1 < 4,
                                     wid + (rep + 1) * NW < NUM_CLS)
            carry = tuple(jnp.zeros((16,), jnp.float32) for _ in range(8))
            for j in range(4):
                @pl.when(cls_ok)
                def _():
                    wait_chunk(m_hbm, rep, j)

                def row_body(i, c):
                    c = list(c)
                    for u in range(8):
                        for gg in range(8):
                            c[gg] = c[gg] + jnp.maximum(
                                rows_v[j * 128 + i * 8 + u,
                                       pl.ds(gg * 16, 16)], 0.0)
                    return tuple(c)

                carry = lax.fori_loop(0, 16, row_body, carry)

                if j == 3:
                    carry = tuple(
                        carry[gg] - 12.0 * jnp.maximum(
                            rows_v[3 * 128 + 115, pl.ds(gg * 16, 16)], 0.0)
                        for gg in range(8))

                @pl.when(nxt_ok)
                def _():
                    fire(m_hbm, rep1, j)

            @pl.when(cls_ok)
            def _():
                for gg in range(8):
                    anch_v[pl.ds(gg * 16, 16)] = carry[gg]
                pltpu.sync_copy(anch_v, anch_hbm.at[bank, cls])
            return 0

        lax.fori_loop(0, 4, rep_body, 0)


def _sc_anchors(cidx, memory_s, memory_t):
    mesh = plsc.VectorSubcoreMesh(core_axis_name="c", subcore_axis_name="s")
    kfn = pl.kernel(
        _sc_anchors_body,
        out_type=jax.ShapeDtypeStruct((2, NUM_CLS, FEAT), jnp.float32),
        mesh=mesh,
        scratch_types=[
            pltpu.VMEM((4, 4, 128), jnp.int32),
            pltpu.VMEM((512, FEAT), jnp.float32),
            pltpu.VMEM((FEAT,), jnp.float32),
            pltpu.SemaphoreType.DMA,
        ],
    )
    return kfn(cidx, memory_s, memory_t)


def _finalize_body(sums_ref, pos_ref, anch_ref, es_ref, et_ref, ccd_ref, rel_ref):
    n_neg_c = (K_TOT - P_POS) * (1.0 / N_DATA) + EPS

    pmask = (lax.broadcasted_iota(jnp.int32, (NW, A_T * 16), 1) % 16) < P_POS

    def closs(bank):
        z = jnp.sum(sums_ref[bank]) * (float(N_DATA) / (BSZ * K_TOT))
        pn = pos_ref[bank] / z
        terms = jnp.log(pn / (pn + n_neg_c))
        return -jnp.sum(jnp.where(pmask, terms, 0.0)) / BSZ

    ccd_ref[...] = jnp.reshape(closs(0) + closs(1), (1, 1))

    def relation(emb, bank):
        a = anch_ref[bank] * (1.0 / PER_CLS)
        a = a * lax.rsqrt(jnp.sum(a * a, axis=1, keepdims=True))
        return lax.dot_general(emb, a, (((1,), (1,)), ((), ())),
                               preferred_element_type=jnp.float32,
                               precision=_HI)

    s_rel = relation(es_ref[...], 0)
    t_rel = relation(et_ref[...], 1)

    def logsoftmax(x):
        m = jnp.max(x, axis=1, keepdims=True)
        s = x - m
        return s - jnp.log(jnp.sum(jnp.exp(s), axis=1, keepdims=True))

    log_p_s = logsoftmax(s_rel)
    log_p_t = logsoftmax(t_rel)
    p_t = jnp.exp(log_p_t)
    rel_ref[...] = jnp.reshape(jnp.sum(p_t * (log_p_t - log_p_s)) * (1.0 / BSZ),
                               (1, 1))


def _finalize(sums, pos, anch, emb_s, emb_t):
    return pl.pallas_call(
        _finalize_body,
        out_shape=(jax.ShapeDtypeStruct((1, 1), jnp.float32),
                   jax.ShapeDtypeStruct((1, 1), jnp.float32)),
    )(sums, pos, anch, emb_s, emb_t)


def kernel(f_s, f_t, batch_label, class_index, num_pos, contrast_idx,
           W_s, b_s, W_t, b_t, memory_s, memory_t):
    ci32 = class_index.astype(jnp.int32)
    cidx = jnp.concatenate(
        [ci32, jnp.broadcast_to(ci32[:, -1:], (NUM_CLS, 12))], axis=1)
    cidx = cidx.reshape(NUM_CLS, 4, 128)
    anch = _sc_anchors(cidx, memory_s, memory_t)

    emb_s, emb_t = _embed(f_s, W_s, b_s, f_t, W_t, b_t)
    spk = _scores(memory_s, memory_t, emb_s, emb_t)

    r = contrast_idx.astype(jnp.int32)
    b = jnp.arange(BSZ, dtype=jnp.int32)[:, None]
    flat = ((r // R_BLK) * (R_BLK * BSZ) + (b // 128) * (R_BLK * 128)
            + (r % R_BLK) * 128 + (b % 128))
    flat = flat.reshape(NW, E_T)
    flat = jnp.pad(flat, ((0, 0), (0, 128 - TAIL))).reshape(NW, N_CHUNK_FULL + 1, 128)

    sums, pos = _sc_scores(
        spk.reshape(S_ROWS * 128), flat)

    ccd, rel = _finalize(sums, pos, anch, emb_s, emb_t)
    return (ccd[0, 0], rel[0, 0])
